# baseline (speedup 1.0000x reference)
import jax
import jax.numpy as jnp
from jax.experimental import pallas as pl
from jax.experimental.pallas import tpu as pltpu

_B, _T, _D = 2, 256, 1024
_E = 8
_FF = 4096
_EPS = 1e-5
_NTOK = _B * _T
_FBLK = 1024
_NF = _FF // _FBLK


def _fiota(shape, dim):
    return jax.lax.broadcasted_iota(jnp.int32, shape, dim).astype(jnp.float32)


def _ln(h, g, b):
    m = jnp.mean(h, axis=-1, keepdims=True)
    v = jnp.mean((h - m) ** 2, axis=-1, keepdims=True)
    return (h - m) / jnp.sqrt(v + _EPS) * g + b


def _fused_kernel(x_ref, fb_ref, kw_ref, kb_ref, vw_ref, vb_ref,
                  ow_ref, ob_ref, gw_ref, gb_ref, g1_ref, b1_ref,
                  w1_ref, b1e_ref, w2_ref, b2e_ref, g2_ref, b2v_ref,
                  out_ref, x1_s, x1b_s, ff_s, w_s):
    k = pl.program_id(0)
    f = pl.program_id(1)

    @pl.when((k == 0) & (f == 0))
    def _():
        xb = x_ref[...]
        K = jnp.dot(xb, kw_ref[...], preferred_element_type=jnp.float32) + kb_ref[...]
        V = jnp.dot(xb, vw_ref[...], preferred_element_type=jnp.float32) + vb_ref[...]
        KV = K * V
        wvs = []
        for b in range(_B):
            Vb = V[b * _T:(b + 1) * _T, :]
            t1 = jnp.sum(KV[b * _T:(b + 1) * _T, :], axis=0, keepdims=True)
            t2 = jnp.dot(fb_ref[b], Vb, preferred_element_type=jnp.float32)
            wvs.append(t2 + t1)
        wv = jnp.concatenate(wvs, axis=0)
        attn = jnp.dot(wv, ow_ref[...], preferred_element_type=jnp.float32) + ob_ref[...]
        x1 = _ln(xb + attn, g1_ref[...], b1_ref[...])
        x1_s[...] = x1
        x1b_s[...] = x1.astype(jnp.bfloat16)
        ff_s[...] = jnp.zeros_like(ff_s)

        logits = jnp.dot(x1, gw_ref[...], preferred_element_type=jnp.float32) + gb_ref[...]
        mx = jnp.max(logits, axis=1, keepdims=True)
        ex = jnp.exp(logits - mx)
        sc = ex / jnp.sum(ex, axis=1, keepdims=True)
        eidx = _fiota((_NTOK, _E), 1)
        m1 = jnp.max(sc, axis=1, keepdims=True)
        e1 = jnp.min(jnp.where(sc == m1, eidx, _E), axis=1, keepdims=True)
        scm = jnp.where(eidx == e1, -jnp.inf, sc)
        m2 = jnp.max(scm, axis=1, keepdims=True)
        e2 = jnp.min(jnp.where(scm == m2, eidx, _E), axis=1, keepdims=True)
        w_s[...] = (jnp.where(eidx == e1, m1, 0.0)
                    + jnp.where(eidx == e2, m2, 0.0))

    @pl.when(k > 0)
    def _():
        h = jnp.maximum(
            jnp.dot(x1b_s[...], w1_ref[0].astype(jnp.bfloat16),
                    preferred_element_type=jnp.float32)
            + b1e_ref[0], 0.0)
        part = jnp.dot(h.astype(jnp.bfloat16), w2_ref[0].astype(jnp.bfloat16),
                       preferred_element_type=jnp.float32)
        lane = jax.lax.broadcasted_iota(jnp.int32, (_NTOK, _E), 1)
        wcol = jnp.sum(jnp.where(lane == k - 1, w_s[...], 0.0),
                       axis=1, keepdims=True)
        contrib = wcol * part

        @pl.when(f == _NF - 1)
        def _():
            ff_s[...] = ff_s[...] + contrib + wcol * b2e_ref[0]

        @pl.when(f != _NF - 1)
        def _():
            ff_s[...] = ff_s[...] + contrib

    @pl.when((k == _E) & (f == _NF - 1))
    def _():
        out_ref[...] = _ln(x1_s[...] + ff_s[...], g2_ref[...], b2v_ref[...])


def kernel(x, fourier_bias, key_w, key_b, value_w, value_b, out_w, out_b,
           gate_w, gate_b, e_w1, e_b1, e_w2, e_b2, ln1_g, ln1_b, ln2_g, ln2_b):
    f32 = jnp.float32
    row = lambda a: a.reshape(1, -1)
    const2 = lambda k, f: (0, 0)
    const3 = lambda k, f: (0, 0, 0)
    eix = lambda k: jnp.maximum(k - 1, 0)

    x2 = pl.pallas_call(
        _fused_kernel,
        grid=(_E + 1, _NF),
        in_specs=[
            pl.BlockSpec((_NTOK, _D), const2),
            pl.BlockSpec((_B, _T, _T), const3),
            pl.BlockSpec((_D, _D), const2),
            pl.BlockSpec((1, _D), const2),
            pl.BlockSpec((_D, _D), const2),
            pl.BlockSpec((1, _D), const2),
            pl.BlockSpec((_D, _D), const2),
            pl.BlockSpec((1, _D), const2),
            pl.BlockSpec((_D, _E), const2),
            pl.BlockSpec((1, _E), const2),
            pl.BlockSpec((1, _D), const2),
            pl.BlockSpec((1, _D), const2),
            pl.BlockSpec((1, _D, _FBLK), lambda k, f: (eix(k), 0, f)),
            pl.BlockSpec((1, 1, _FBLK), lambda k, f: (eix(k), 0, f)),
            pl.BlockSpec((1, _FBLK, _D), lambda k, f: (eix(k), f, 0)),
            pl.BlockSpec((1, 1, _D), lambda k, f: (eix(k), 0, 0)),
            pl.BlockSpec((1, _D), const2),
            pl.BlockSpec((1, _D), const2),
        ],
        out_specs=pl.BlockSpec((_NTOK, _D), const2),
        out_shape=jax.ShapeDtypeStruct((_NTOK, _D), f32),
        scratch_shapes=[
            pltpu.VMEM((_NTOK, _D), f32),
            pltpu.VMEM((_NTOK, _D), jnp.bfloat16),
            pltpu.VMEM((_NTOK, _D), f32),
            pltpu.VMEM((_NTOK, _E), f32),
        ],
    )(x.reshape(_NTOK, _D), fourier_bias, key_w, row(key_b),
      value_w, row(value_b), out_w, row(out_b), gate_w, row(gate_b),
      row(ln1_g), row(ln1_b),
      e_w1, e_b1.reshape(_E, 1, _FF), e_w2, e_b2.reshape(_E, 1, _D),
      row(ln2_g), row(ln2_b))

    return x2.reshape(_B, _T, _D)


# speedup vs baseline: 1.3094x; 1.3094x over previous
import jax
import jax.numpy as jnp
from jax.experimental import pallas as pl
from jax.experimental.pallas import tpu as pltpu

_B, _T, _D = 2, 256, 1024
_E = 8
_FF = 4096
_EPS = 1e-5
_NTOK = _B * _T
_BLK = 256
_JB = 2
_NR = _E * _JB
_FBLK = 1024
_NF = _FF // _FBLK
_NBUF = 3


def _fiota(shape, dim):
    return jax.lax.broadcasted_iota(jnp.int32, shape, dim).astype(jnp.float32)


def _ln(h, g, b):
    m = jnp.mean(h, axis=-1, keepdims=True)
    v = jnp.mean((h - m) ** 2, axis=-1, keepdims=True)
    return (h - m) / jnp.sqrt(v + _EPS) * g + b


def _fused_kernel(x_ref, fb_ref, kw_ref, kb_ref, vw_ref, vb_ref,
                  ow_ref, ob_ref, gw_ref, gb_ref, g1_ref, b1_ref,
                  w1_hbm, b1e_ref, w2_hbm, b2e_ref, g2_ref, b2v_ref,
                  out_ref,
                  w1b, w2b, x1b_s, ff_s, xg_s, acc_s, st_s, sw_s, nb_s,
                  sems):
    tiles = [(e, f) for e in range(_E) for f in range(_NF)]

    def start_copy(i):
        e, f = tiles[i]
        s = i % _NBUF
        pltpu.make_async_copy(
            w1_hbm.at[e, :, f * _FBLK:(f + 1) * _FBLK], w1b.at[s],
            sems.at[0, s]).start()
        pltpu.make_async_copy(
            w2_hbm.at[e, f * _FBLK:(f + 1) * _FBLK, :], w2b.at[s],
            sems.at[1, s]).start()

    def wait_copy(i):
        e, f = tiles[i]
        s = i % _NBUF
        pltpu.make_async_copy(
            w1_hbm.at[e, :, f * _FBLK:(f + 1) * _FBLK], w1b.at[s],
            sems.at[0, s]).wait()
        pltpu.make_async_copy(
            w2_hbm.at[e, f * _FBLK:(f + 1) * _FBLK, :], w2b.at[s],
            sems.at[1, s]).wait()

    for i in range(_NBUF):
        start_copy(i)

    xb = x_ref[...]
    K = jnp.dot(xb, kw_ref[...], preferred_element_type=jnp.float32) + kb_ref[...]
    V = jnp.dot(xb, vw_ref[...], preferred_element_type=jnp.float32) + vb_ref[...]
    KV = K * V
    wvs = []
    for b in range(_B):
        Vb = V[b * _T:(b + 1) * _T, :]
        t1 = jnp.sum(KV[b * _T:(b + 1) * _T, :], axis=0, keepdims=True)
        t2 = jnp.dot(fb_ref[b], Vb, preferred_element_type=jnp.float32)
        wvs.append(t2 + t1)
    wv = jnp.concatenate(wvs, axis=0)
    attn = jnp.dot(wv, ow_ref[...], preferred_element_type=jnp.float32) + ob_ref[...]
    x1 = _ln(xb + attn, g1_ref[...], b1_ref[...])
    out_ref[...] = x1
    x1b_s[...] = x1.astype(jnp.bfloat16)
    ff_s[...] = jnp.zeros_like(ff_s)

    logits = jnp.dot(x1, gw_ref[...], preferred_element_type=jnp.float32) + gb_ref[...]
    mx = jnp.max(logits, axis=1, keepdims=True)
    ex = jnp.exp(logits - mx)
    sc = ex / jnp.sum(ex, axis=1, keepdims=True)
    eidx = _fiota((_NTOK, _E), 1)
    m1 = jnp.max(sc, axis=1, keepdims=True)
    e1 = jnp.min(jnp.where(sc == m1, eidx, _E), axis=1, keepdims=True)
    scm = jnp.where(eidx == e1, -jnp.inf, sc)
    m2 = jnp.max(scm, axis=1, keepdims=True)
    e2 = jnp.min(jnp.where(scm == m2, eidx, _E), axis=1, keepdims=True)

    oh1 = (eidx == e1).astype(jnp.float32)
    oh2 = (eidx == e2).astype(jnp.float32)
    mask = oh1 + oh2
    ii = _fiota((_NTOK, _NTOK), 0)
    jj = _fiota((_NTOK, _NTOK), 1)
    ltri = (jj < ii).astype(jnp.float32)
    rank = jnp.dot(ltri, mask, preferred_element_type=jnp.float32)
    counts = jnp.sum(mask, axis=0, keepdims=True)
    nb_s[...] = jnp.floor((counts + (_BLK - 1)) * (1.0 / _BLK)).astype(jnp.int32)
    base = _BLK * _JB * _fiota((_NTOK, _E), 1)
    slotv = base + rank
    slot1 = jnp.sum(oh1 * slotv, axis=1, keepdims=True)
    slot2 = jnp.sum(oh2 * slotv, axis=1, keepdims=True)
    tcol = _fiota((_NTOK, _BLK), 0)
    lidx = _fiota((_NTOK, _BLK), 1)
    for r in range(_NR):
        srow = lidx + (r * _BLK)
        m1s = slot1 == srow
        m2s = slot2 == srow
        st_s[r:r + 1, :] = jnp.sum(
            jnp.where(m1s, tcol, 0.0) + jnp.where(m2s, tcol, 0.0),
            axis=0, keepdims=True).astype(jnp.int32)
        sw_s[r:r + 1, :] = jnp.sum(
            jnp.where(m1s, m1, 0.0) + jnp.where(m2s, m2, 0.0),
            axis=0, keepdims=True)

    gcols = jax.lax.broadcasted_iota(jnp.int32, (_BLK, _NTOK), 1)
    srows = jax.lax.broadcasted_iota(jnp.int32, (_NTOK, _BLK), 0)
    for i, (e, f) in enumerate(tiles):
        wait_copy(i)
        s = i % _NBUF
        w1t = w1b[s].astype(jnp.bfloat16)
        w2t = w2b[s].astype(jnp.bfloat16)
        b1t = b1e_ref[e, 0, f * _FBLK:(f + 1) * _FBLK]
        for j in range(_JB):
            r = _JB * e + j

            @pl.when(nb_s[0, e] > j)
            def _(r=r, j=j, f=f, e=e, w1t=w1t, w2t=w2t, b1t=b1t):
                toks = st_s[r]
                if f == 0:
                    P = (toks[:, None] == gcols).astype(jnp.bfloat16)
                    xg_s[j] = jnp.dot(P, x1b_s[...],
                                      preferred_element_type=jnp.float32
                                      ).astype(jnp.bfloat16)
                h = jnp.maximum(
                    jnp.dot(xg_s[j], w1t, preferred_element_type=jnp.float32)
                    + b1t, 0.0)
                part = jnp.dot(h.astype(jnp.bfloat16), w2t,
                               preferred_element_type=jnp.float32)
                if f == 0:
                    acc_s[j] = part
                else:
                    acc_s[j] = acc_s[j] + part
                if f == _NF - 1:
                    h2 = acc_s[j] + b2e_ref[e, 0, :]
                    contrib = (h2 * sw_s[r][:, None]).astype(jnp.bfloat16)
                    Pt = (srows == toks[None, :]).astype(jnp.bfloat16)
                    ff_s[...] = ff_s[...] + jnp.dot(
                        Pt, contrib, preferred_element_type=jnp.float32)
        if i + _NBUF < len(tiles):
            start_copy(i + _NBUF)

    out_ref[...] = _ln(out_ref[...] + ff_s[...], g2_ref[...], b2v_ref[...])


def kernel(x, fourier_bias, key_w, key_b, value_w, value_b, out_w, out_b,
           gate_w, gate_b, e_w1, e_b1, e_w2, e_b2, ln1_g, ln1_b, ln2_g, ln2_b):
    f32 = jnp.float32
    row = lambda a: a.reshape(1, -1)
    vm = pl.BlockSpec(memory_space=pltpu.VMEM)
    anym = pl.BlockSpec(memory_space=pl.ANY)

    x2 = pl.pallas_call(
        _fused_kernel,
        in_specs=[vm, vm, vm, vm, vm, vm, vm, vm, vm, vm, vm, vm,
                  anym, vm, anym, vm, vm, vm],
        out_specs=vm,
        out_shape=jax.ShapeDtypeStruct((_NTOK, _D), f32),
        scratch_shapes=[
            pltpu.VMEM((_NBUF, _D, _FBLK), f32),
            pltpu.VMEM((_NBUF, _FBLK, _D), f32),
            pltpu.VMEM((_NTOK, _D), jnp.bfloat16),
            pltpu.VMEM((_NTOK, _D), f32),
            pltpu.VMEM((_JB, _BLK, _D), jnp.bfloat16),
            pltpu.VMEM((_JB, _BLK, _D), f32),
            pltpu.VMEM((_NR, _BLK), jnp.int32),
            pltpu.VMEM((_NR, _BLK), f32),
            pltpu.VMEM((1, _E), jnp.int32),
            pltpu.SemaphoreType.DMA((2, _NBUF)),
        ],
    )(x.reshape(_NTOK, _D), fourier_bias, key_w, row(key_b),
      value_w, row(value_b), out_w, row(out_b), gate_w, row(gate_b),
      row(ln1_g), row(ln1_b),
      e_w1, e_b1.reshape(_E, 1, _FF), e_w2, e_b2.reshape(_E, 1, _D),
      row(ln2_g), row(ln2_b))

    return x2.reshape(_B, _T, _D)


# speedup vs baseline: 1.3104x; 1.0008x over previous
import jax
import jax.numpy as jnp
from jax.experimental import pallas as pl
from jax.experimental.pallas import tpu as pltpu

_B, _T, _D = 2, 256, 1024
_E = 8
_FF = 4096
_EPS = 1e-5
_NTOK = _B * _T
_BLK = 256
_JB = 2
_NR = _E * _JB
_FBLK = 1024
_NF = _FF // _FBLK
_NBUF = 3


def _fiota(shape, dim):
    return jax.lax.broadcasted_iota(jnp.int32, shape, dim).astype(jnp.float32)


def _ln(h, g, b):
    m = jnp.mean(h, axis=-1, keepdims=True)
    v = jnp.mean((h - m) ** 2, axis=-1, keepdims=True)
    return (h - m) / jnp.sqrt(v + _EPS) * g + b


def _fused_kernel(x_ref, fb_ref, kw_ref, kb_ref, vw_ref, vb_ref,
                  ow_ref, ob_ref, gw_ref, gb_ref, g1_ref, b1_ref,
                  w1_hbm, b1e_ref, w2_hbm, b2e_ref, g2_ref, b2v_ref,
                  out_ref,
                  w1b, w2b, x1b_s, ff_s, xg_s, acc_s, st_s, sw_s, nb_s,
                  sems):
    tiles = [(e, f) for e in range(_E) for f in range(_NF)]

    _H = _D // 2

    def _copies(i):
        e, f = tiles[i]
        s = i % _NBUF
        c = []
        for p in range(2):
            rs = slice(p * _H, (p + 1) * _H)
            c.append(pltpu.make_async_copy(
                w1_hbm.at[e, rs, f * _FBLK:(f + 1) * _FBLK], w1b.at[s, rs],
                sems.at[2 * p, s]))
            c.append(pltpu.make_async_copy(
                w2_hbm.at[e, f * _FBLK + p * _H:f * _FBLK + (p + 1) * _H, :],
                w2b.at[s, p * _H:(p + 1) * _H], sems.at[2 * p + 1, s]))
        return c

    def start_copy(i):
        for c in _copies(i):
            c.start()

    def wait_copy(i):
        for c in _copies(i):
            c.wait()

    for i in range(_NBUF):
        start_copy(i)

    xb = x_ref[...]
    K = jnp.dot(xb, kw_ref[...], preferred_element_type=jnp.float32) + kb_ref[...]
    V = jnp.dot(xb, vw_ref[...], preferred_element_type=jnp.float32) + vb_ref[...]
    KV = K * V
    wvs = []
    for b in range(_B):
        Vb = V[b * _T:(b + 1) * _T, :]
        t1 = jnp.sum(KV[b * _T:(b + 1) * _T, :], axis=0, keepdims=True)
        t2 = jnp.dot(fb_ref[b], Vb, preferred_element_type=jnp.float32)
        wvs.append(t2 + t1)
    wv = jnp.concatenate(wvs, axis=0)
    attn = jnp.dot(wv, ow_ref[...], preferred_element_type=jnp.float32) + ob_ref[...]
    x1 = _ln(xb + attn, g1_ref[...], b1_ref[...])
    out_ref[...] = x1
    x1b_s[...] = x1.astype(jnp.bfloat16)
    ff_s[...] = jnp.zeros_like(ff_s)

    logits = jnp.dot(x1, gw_ref[...], preferred_element_type=jnp.float32) + gb_ref[...]
    mx = jnp.max(logits, axis=1, keepdims=True)
    ex = jnp.exp(logits - mx)
    sc = ex / jnp.sum(ex, axis=1, keepdims=True)
    eidx = _fiota((_NTOK, _E), 1)
    m1 = jnp.max(sc, axis=1, keepdims=True)
    e1 = jnp.min(jnp.where(sc == m1, eidx, _E), axis=1, keepdims=True)
    scm = jnp.where(eidx == e1, -jnp.inf, sc)
    m2 = jnp.max(scm, axis=1, keepdims=True)
    e2 = jnp.min(jnp.where(scm == m2, eidx, _E), axis=1, keepdims=True)

    oh1 = (eidx == e1).astype(jnp.float32)
    oh2 = (eidx == e2).astype(jnp.float32)
    mask = oh1 + oh2
    ii = _fiota((_NTOK, _NTOK), 0)
    jj = _fiota((_NTOK, _NTOK), 1)
    ltri = (jj < ii).astype(jnp.float32)
    rank = jnp.dot(ltri, mask, preferred_element_type=jnp.float32)
    counts = jnp.sum(mask, axis=0, keepdims=True)
    nb_s[...] = jnp.floor((counts + (_BLK - 1)) * (1.0 / _BLK)).astype(jnp.int32)
    base = _BLK * _JB * _fiota((_NTOK, _E), 1)
    slotv = base + rank
    slot1 = jnp.sum(oh1 * slotv, axis=1, keepdims=True)
    slot2 = jnp.sum(oh2 * slotv, axis=1, keepdims=True)
    tcol = _fiota((_NTOK, _BLK), 0)
    lidx = _fiota((_NTOK, _BLK), 1)
    for r in range(_NR):
        srow = lidx + (r * _BLK)
        m1s = slot1 == srow
        m2s = slot2 == srow
        st_s[r:r + 1, :] = jnp.sum(
            jnp.where(m1s, tcol, 0.0) + jnp.where(m2s, tcol, 0.0),
            axis=0, keepdims=True).astype(jnp.int32)
        sw_s[r:r + 1, :] = jnp.sum(
            jnp.where(m1s, m1, 0.0) + jnp.where(m2s, m2, 0.0),
            axis=0, keepdims=True)

    gcols = jax.lax.broadcasted_iota(jnp.int32, (_BLK, _NTOK), 1)
    srows = jax.lax.broadcasted_iota(jnp.int32, (_NTOK, _BLK), 0)
    for i, (e, f) in enumerate(tiles):
        wait_copy(i)
        s = i % _NBUF
        w1t = w1b[s].astype(jnp.bfloat16)
        w2t = w2b[s].astype(jnp.bfloat16)
        b1t = b1e_ref[e, 0, f * _FBLK:(f + 1) * _FBLK]
        for j in range(_JB):
            r = _JB * e + j

            @pl.when(nb_s[0, e] > j)
            def _(r=r, j=j, f=f, e=e, w1t=w1t, w2t=w2t, b1t=b1t):
                toks = st_s[r]
                if f == 0:
                    P = (toks[:, None] == gcols).astype(jnp.bfloat16)
                    xg_s[j] = jnp.dot(P, x1b_s[...],
                                      preferred_element_type=jnp.float32
                                      ).astype(jnp.bfloat16)
                h = jnp.maximum(
                    jnp.dot(xg_s[j], w1t, preferred_element_type=jnp.float32)
                    + b1t, 0.0)
                part = jnp.dot(h.astype(jnp.bfloat16), w2t,
                               preferred_element_type=jnp.float32)
                if f == 0:
                    acc_s[j] = part
                else:
                    acc_s[j] = acc_s[j] + part
                if f == _NF - 1:
                    h2 = acc_s[j] + b2e_ref[e, 0, :]
                    contrib = (h2 * sw_s[r][:, None]).astype(jnp.bfloat16)
                    Pt = (srows == toks[None, :]).astype(jnp.bfloat16)
                    ff_s[...] = ff_s[...] + jnp.dot(
                        Pt, contrib, preferred_element_type=jnp.float32)
        if i + _NBUF < len(tiles):
            start_copy(i + _NBUF)

    out_ref[...] = _ln(out_ref[...] + ff_s[...], g2_ref[...], b2v_ref[...])


def kernel(x, fourier_bias, key_w, key_b, value_w, value_b, out_w, out_b,
           gate_w, gate_b, e_w1, e_b1, e_w2, e_b2, ln1_g, ln1_b, ln2_g, ln2_b):
    f32 = jnp.float32
    row = lambda a: a.reshape(1, -1)
    vm = pl.BlockSpec(memory_space=pltpu.VMEM)
    anym = pl.BlockSpec(memory_space=pl.ANY)

    x2 = pl.pallas_call(
        _fused_kernel,
        in_specs=[vm, vm, vm, vm, vm, vm, vm, vm, vm, vm, vm, vm,
                  anym, vm, anym, vm, vm, vm],
        out_specs=vm,
        out_shape=jax.ShapeDtypeStruct((_NTOK, _D), f32),
        scratch_shapes=[
            pltpu.VMEM((_NBUF, _D, _FBLK), f32),
            pltpu.VMEM((_NBUF, _FBLK, _D), f32),
            pltpu.VMEM((_NTOK, _D), jnp.bfloat16),
            pltpu.VMEM((_NTOK, _D), f32),
            pltpu.VMEM((_JB, _BLK, _D), jnp.bfloat16),
            pltpu.VMEM((_JB, _BLK, _D), f32),
            pltpu.VMEM((_NR, _BLK), jnp.int32),
            pltpu.VMEM((_NR, _BLK), f32),
            pltpu.VMEM((1, _E), jnp.int32),
            pltpu.SemaphoreType.DMA((4, _NBUF)),
        ],
    )(x.reshape(_NTOK, _D), fourier_bias, key_w, row(key_b),
      value_w, row(value_b), out_w, row(out_b), gate_w, row(gate_b),
      row(ln1_g), row(ln1_b),
      e_w1, e_b1.reshape(_E, 1, _FF), e_w2, e_b2.reshape(_E, 1, _D),
      row(ln2_g), row(ln2_b))

    return x2.reshape(_B, _T, _D)
